# Initial kernel scaffold; baseline (speedup 1.0000x reference)
import jax
import jax.numpy as jnp
from jax.experimental import pallas as pl


def kernel(discrete_0, table_d0, discrete_1, table_d1, discrete_2, table_d2, discrete_3, table_d3, discrete_4, table_d4, discrete_5, table_d5, entity, table_e, cont0_values, cont0_indicators, W_cont0, b_cont0, cont1_values, cont1_indicators, W_cont1, b_cont1, temp_values, temp_indicators, W_temp, b_temp):
    raise NotImplementedError("write your pallas kernel here")


# speedup vs baseline: 1.0454x; 1.0454x over previous
import math

import jax
import jax.numpy as jnp
from jax import lax
from jax.experimental import pallas as pl
from jax.experimental.pallas import tpu as pltpu
from jax.experimental.pallas import tpu_sc as plsc

N, L, D = 1024, 50, 64
NT = N * L
N_BANDS, OFFSET = 8, 3
NWF = N_BANDS + OFFSET + 1
C = 10

NC, NS = 2, 16
NWORK = NC * NS
TPW = NT // NWORK
TCH = 80
NFULL = TPW // TCH

T2 = 2048

_GATHER_CH = (0, 1, 2, 3, 4, 5, 8)
_FOUR_CH = (6, 7, 9)


def _fourier_tc(v0, i0, v1, i1, v2, i2, W0, b0, W1, b1, W2, b2, wrow):
    def body(v0r, i0r, v1r, i1r, v2r, i2r, W0r, b0r, W1r, b1r, W2r, b2r,
             wr, o0r, o1r, o2r):
        lane = lax.broadcasted_iota(jnp.int32, (T2, 2 * NWF), 1)
        for vr, ir, Wr, br, outr in ((v0r, i0r, W0r, b0r, o0r),
                                     (v1r, i1r, W1r, b1r, o1r),
                                     (v2r, i2r, W2r, b2r, o2r)):
            x = vr[...] - ir[...]
            a = x * wr[...]
            f = jnp.where(lane < NWF, jnp.sin(a), jnp.cos(a))
            outr[...] = (jnp.dot(f, Wr[...], preferred_element_type=jnp.float32)
                         + br[...])

    col = pl.BlockSpec((T2, 1), lambda i: (i, 0))
    full2 = pl.BlockSpec((2 * NWF, D), lambda i: (0, 0))
    fullb = pl.BlockSpec((1, D), lambda i: (0, 0))
    fullw = pl.BlockSpec((1, 2 * NWF), lambda i: (0, 0))
    outspec = pl.BlockSpec((T2, D), lambda i: (i, 0))
    return pl.pallas_call(
        body,
        grid=(NT // T2,),
        in_specs=[col, col, col, col, col, col,
                  full2, fullb, full2, fullb, full2, fullb, fullw],
        out_specs=[outspec, outspec, outspec],
        out_shape=[jax.ShapeDtypeStruct((NT, D), jnp.float32)] * 3,
    )(v0, i0, v1, i1, v2, i2, W0, b0, W1, b1, W2, b2, wrow)


def _sc_interleave(tables, idx_stack, f0, f1, f2):
    mesh = plsc.VectorSubcoreMesh(core_axis_name="c", subcore_axis_name="s",
                                  num_cores=NC, num_subcores=NS)

    import functools

    @functools.partial(
        pl.kernel,
        out_type=jax.ShapeDtypeStruct((NT, C, D), jnp.float32),
        mesh=mesh,
        scratch_types=[pltpu.VMEM((TPW,), jnp.int32) for _ in range(7)]
        + [pltpu.VMEM((TCH, 1, D), jnp.float32) for _ in range(10)]
        + [pltpu.SemaphoreType.DMA, pltpu.SemaphoreType.DMA],
    )
    def k(t0, t1, t2, t3, t4, t5, te, idx_hbm, f0h, f1h, f2h, out,
          i0, i1, i2, i3, i4, i5, i6,
          r0, r1, r2, r3, r4, r5, r6, r7, r8, r9, gsem, wsem):
        tabs = (t0, t1, t2, t3, t4, t5, te)
        idxv = (i0, i1, i2, i3, i4, i5, i6)
        rbufs = (r0, r1, r2, r3, r4, r5, r6)
        fbufs = (r7, r8, r9)
        fhs = (f0h, f1h, f2h)
        wid = lax.axis_index("s") * NC + lax.axis_index("c")
        wbase = pl.multiple_of(wid * TPW, TPW)
        for ci in range(7):
            pltpu.sync_copy(idx_hbm.at[pl.ds(ci * NT + wbase, TPW)],
                            idxv[ci])

        def chunk(off, t):
            base = pl.multiple_of(wbase + off, 8)
            loads = []
            for ci, (tab, rb) in enumerate(zip(tabs, rbufs)):
                loads.append(pltpu.async_copy(
                    tab.at[idxv[ci].at[pl.ds(off, t)]],
                    rb.at[pl.ds(0, t)], gsem))
            for fh, fb in zip(fhs, fbufs):
                loads.append(pltpu.async_copy(
                    fh.at[pl.ds(base, t)], fb.at[pl.ds(0, t)], gsem))
            for cp in loads:
                cp.wait()
            stores = []
            for ci, ch in enumerate(_GATHER_CH):
                stores.append(pltpu.async_copy(
                    rbufs[ci].at[pl.ds(0, t)],
                    out.at[pl.ds(base, t), pl.ds(ch, 1)], wsem))
            for fi, ch in enumerate(_FOUR_CH):
                stores.append(pltpu.async_copy(
                    fbufs[fi].at[pl.ds(0, t)],
                    out.at[pl.ds(base, t), pl.ds(ch, 1)], wsem))
            for cp in stores:
                cp.wait()

        def body(i, _):
            chunk(i * TCH, TCH)
            return 0

        lax.fori_loop(0, NFULL, body, 0)

    return k(*tables, idx_stack, f0, f1, f2)


def kernel(discrete_0, table_d0, discrete_1, table_d1, discrete_2, table_d2,
           discrete_3, table_d3, discrete_4, table_d4, discrete_5, table_d5,
           entity, table_e,
           cont0_values, cont0_indicators, W_cont0, b_cont0,
           cont1_values, cont1_indicators, W_cont1, b_cont1,
           temp_values, temp_indicators, W_temp, b_temp):
    w = (2.0 ** jnp.linspace(-float(N_BANDS), float(OFFSET), NWF)) * math.pi
    wrow = jnp.concatenate([w, w]).reshape(1, 2 * NWF).astype(jnp.float32)

    f0, f1, f2 = _fourier_tc(
        cont0_values.reshape(NT, 1), cont0_indicators.reshape(NT, 1),
        cont1_values.reshape(NT, 1), cont1_indicators.reshape(NT, 1),
        temp_values.reshape(NT, 1), temp_indicators.reshape(NT, 1),
        W_cont0, b_cont0.reshape(1, D), W_cont1, b_cont1.reshape(1, D),
        W_temp, b_temp.reshape(1, D), wrow)

    idx_stack = jnp.stack([
        discrete_0.reshape(NT), discrete_1.reshape(NT),
        discrete_2.reshape(NT), discrete_3.reshape(NT),
        discrete_4.reshape(NT), discrete_5.reshape(NT),
        entity.reshape(NT)]).reshape(7 * NT)

    tables = [t.reshape(t.shape[0], 1, D) for t in
              (table_d0, table_d1, table_d2, table_d3, table_d4, table_d5,
               table_e)]

    out = _sc_interleave(tables, idx_stack,
                         f0.reshape(NT, 1, D), f1.reshape(NT, 1, D),
                         f2.reshape(NT, 1, D))
    return out.reshape(N, L, C, D)


# speedup vs baseline: 1.6666x; 1.5942x over previous
import functools
import math

import jax
import jax.numpy as jnp
from jax import lax
from jax.experimental import pallas as pl
from jax.experimental.pallas import tpu as pltpu
from jax.experimental.pallas import tpu_sc as plsc

N, L, D = 1024, 50, 64
NT = N * L
N_BANDS, OFFSET = 8, 3
NWF = N_BANDS + OFFSET + 1
C = 10
VD = 100004
VE = 2052
OUTSZ = L * C * D * N

NC, NS = 2, 16
NWORK = NC * NS

LCH = 5
TQ = LCH * N
NQ = L // LCH

G = 16
TBLK = G * 128

_W0 = math.pi * (2.0 ** float(-N_BANDS))


def _fourier_tc(v0, i0, v1, i1, v2, i2, Wbig, b0, b1, b2):
    def body(v0r, i0r, v1r, i1r, v2r, i2r, Wr, b0r, b1r, b2r, o0r, o1r, o2r):
        rows = []
        for vr, ir in ((v0r, i0r), (v1r, i1r), (v2r, i2r)):
            x = vr[...] - ir[...]
            angles = [x * (_W0 * 2.0 ** k) for k in range(NWF)]
            rows.extend([jnp.sin(a) for a in angles])
            rows.extend([jnp.cos(a) for a in angles])
        S = jnp.stack(rows)
        W = Wr[...]
        for r in range(G):
            o = lax.dot_general(W, S[:, r, :], (((0,), (0,)), ((), ())),
                                preferred_element_type=jnp.float32)
            sl = pl.ds(r * 128, 128)
            o0r[:, sl] = o[0:D, :] + b0r[...]
            o1r[:, sl] = o[D:2 * D, :] + b1r[...]
            o2r[:, sl] = o[2 * D:3 * D, :] + b2r[...]

    blk = pl.BlockSpec((G, 128), lambda i: (i, 0))
    wspec = pl.BlockSpec((6 * NWF, 3 * D), lambda i: (0, 0))
    bspec = pl.BlockSpec((D, 1), lambda i: (0, 0))
    outspec = pl.BlockSpec((D, TBLK), lambda i: (0, i))
    return pl.pallas_call(
        body,
        grid=(NT // TBLK,),
        in_specs=[blk] * 6 + [wspec, bspec, bspec, bspec],
        out_specs=[outspec] * 3,
        out_shape=[jax.ShapeDtypeStruct((D, NT), jnp.float32)] * 3,
    )(v0, i0, v1, i1, v2, i2, Wbig, b0, b1, b2)


def _sc_scatter(tabsT, idxF, foursF):
    mesh = plsc.VectorSubcoreMesh(core_axis_name="c", subcore_axis_name="s",
                                  num_cores=NC, num_subcores=NS)

    @functools.partial(
        pl.kernel,
        out_type=jax.ShapeDtypeStruct((OUTSZ,), jnp.float32),
        mesh=mesh,
        scratch_types=[pltpu.VMEM((VD,), jnp.float32),
                       pltpu.VMEM((VE,), jnp.float32),
                       pltpu.VMEM((TQ,), jnp.int32),
                       pltpu.VMEM((TQ,), jnp.float32),
                       pltpu.SemaphoreType.DMA,
                       pltpu.SemaphoreType.DMA],
        compiler_params=pltpu.CompilerParams(needs_layout_passes=False),
    )
    def k(t0, t1, t2, t3, t4, t5, te, x0, x1, x2, x3, x4, x5, x6,
          f0h, f1h, f2h, out, rowv, rowe, idxq, outq, lsem, wsem):
        tabs = (t0, t1, t2, t3, t4, t5, te)
        idxs = (x0, x1, x2, x3, x4, x5, x6)
        fhs = (f0h, f1h, f2h)
        wid = lax.axis_index("s") * NC + lax.axis_index("c")

        def flush(c, d, q):
            stores = []
            for lr in range(LCH):
                off = ((q * LCH + lr) * C + c) * D * N + d * N
                stores.append(pltpu.async_copy(
                    outq.at[pl.ds(lr * N, N)],
                    out.at[pl.ds(pl.multiple_of(off, N), N)], wsem))
            for cp in stores:
                cp.wait()

        def gather_task(ci, c, d):
            tab, row = tabs[ci], (rowv if ci < 6 else rowe)
            pltpu.async_copy(tab.at[d], row, lsem).wait()

            def qbody(q, _):
                pltpu.async_copy(
                    idxs[ci].at[pl.ds(pl.multiple_of(q * TQ, TQ), TQ)],
                    idxq, lsem).wait()

                def body(i, _):
                    iv = idxq[pl.ds(i * 16, 16)]
                    outq[pl.ds(i * 16, 16)] = plsc.load_gather(row, [iv])
                    return 0

                lax.fori_loop(0, TQ // 16, body, 0)
                flush(c, d, q)
                return 0

            lax.fori_loop(0, NQ, qbody, 0)

        def copy_task(fi, c, d):
            fh = fhs[fi]

            def qbody(q, _):
                pltpu.async_copy(
                    fh.at[pl.ds(pl.multiple_of(d * NT + q * TQ, TQ), TQ)],
                    outq, lsem).wait()
                flush(c, d, q)
                return 0

            lax.fori_loop(0, NQ, qbody, 0)

        def rbody(rep, _):
            d = wid * 2 + rep
            for ci, c in enumerate((0, 1, 2, 3, 4, 5, 8)):
                gather_task(ci, c, d)
            for fi, c in enumerate((6, 7, 9)):
                copy_task(fi, c, d)
            return 0

        lax.fori_loop(0, 2, rbody, 0)

    return k(*tabsT, *idxF, *foursF)


def kernel(discrete_0, table_d0, discrete_1, table_d1, discrete_2, table_d2,
           discrete_3, table_d3, discrete_4, table_d4, discrete_5, table_d5,
           entity, table_e,
           cont0_values, cont0_indicators, W_cont0, b_cont0,
           cont1_values, cont1_indicators, W_cont1, b_cont1,
           temp_values, temp_indicators, W_temp, b_temp):
    Z = jnp.zeros((2 * NWF, D), jnp.float32)
    Wbig = jnp.concatenate([
        jnp.concatenate([W_cont0, Z, Z], axis=1),
        jnp.concatenate([Z, W_cont1, Z], axis=1),
        jnp.concatenate([Z, Z, W_temp], axis=1)], axis=0)

    def tok(a):
        return a.T.reshape(NT // 128, 128)

    f0, f1, f2 = _fourier_tc(
        tok(cont0_values), tok(cont0_indicators),
        tok(cont1_values), tok(cont1_indicators),
        tok(temp_values), tok(temp_indicators),
        Wbig, b_cont0.reshape(D, 1), b_cont1.reshape(D, 1),
        b_temp.reshape(D, 1))

    idxF = [a.T.reshape(NT) for a in
            (discrete_0, discrete_1, discrete_2, discrete_3, discrete_4,
             discrete_5, entity)]
    tabsT = [t.T for t in
             (table_d0, table_d1, table_d2, table_d3, table_d4, table_d5,
              table_e)]
    foursF = [f.reshape(D * NT) for f in (f0, f1, f2)]

    out = _sc_scatter(tabsT, idxF, foursF)
    return out.reshape(L, C, D, N).transpose(3, 0, 1, 2)


# speedup vs baseline: 2.2634x; 1.3581x over previous
import functools
import math

import jax
import jax.numpy as jnp
from jax import lax
from jax.experimental import pallas as pl
from jax.experimental.pallas import tpu as pltpu
from jax.experimental.pallas import tpu_sc as plsc

N, L, D = 1024, 50, 64
NT = N * L
N_BANDS, OFFSET = 8, 3
NWF = N_BANDS + OFFSET + 1
C = 10
VD = 100004
VE = 2052
OUTSZ = L * C * D * N

NC, NS = 2, 16
NWORK = NC * NS

LCH = 5
TQ = LCH * N
NQ = L // LCH

G = 16
TBLK = G * 128

_W0 = math.pi * (2.0 ** float(-N_BANDS))


def _fourier_tc(v0, i0, v1, i1, v2, i2, Wbig, b0, b1, b2):
    def body(v0r, i0r, v1r, i1r, v2r, i2r, Wr, b0r, b1r, b2r, o0r, o1r, o2r):
        rows = []
        for vr, ir in ((v0r, i0r), (v1r, i1r), (v2r, i2r)):
            x = vr[...] - ir[...]
            angles = [x * (_W0 * 2.0 ** k) for k in range(NWF)]
            rows.extend([jnp.sin(a) for a in angles])
            rows.extend([jnp.cos(a) for a in angles])
        S = jnp.stack(rows)
        W = Wr[...]
        for r in range(G):
            o = lax.dot_general(W, S[:, r, :], (((0,), (0,)), ((), ())),
                                preferred_element_type=jnp.float32)
            sl = pl.ds(r * 128, 128)
            o0r[:, sl] = o[0:D, :] + b0r[...]
            o1r[:, sl] = o[D:2 * D, :] + b1r[...]
            o2r[:, sl] = o[2 * D:3 * D, :] + b2r[...]

    blk = pl.BlockSpec((G, 128), lambda i: (i, 0))
    wspec = pl.BlockSpec((6 * NWF, 3 * D), lambda i: (0, 0))
    bspec = pl.BlockSpec((D, 1), lambda i: (0, 0))
    outspec = pl.BlockSpec((D, TBLK), lambda i: (0, i))
    return pl.pallas_call(
        body,
        grid=(NT // TBLK,),
        in_specs=[blk] * 6 + [wspec, bspec, bspec, bspec],
        out_specs=[outspec] * 3,
        out_shape=[jax.ShapeDtypeStruct((D, NT), jnp.float32)] * 3,
    )(v0, i0, v1, i1, v2, i2, Wbig, b0, b1, b2)


def _sc_scatter(tabsT, idxF, foursF):
    mesh = plsc.VectorSubcoreMesh(core_axis_name="c", subcore_axis_name="s",
                                  num_cores=NC, num_subcores=NS)

    @functools.partial(
        pl.kernel,
        out_type=jax.ShapeDtypeStruct((OUTSZ,), jnp.float32),
        mesh=mesh,
        scratch_types=[pltpu.VMEM((VD,), jnp.float32),
                       pltpu.VMEM((VE,), jnp.float32),
                       pltpu.VMEM((TQ,), jnp.int32),
                       pltpu.VMEM((TQ,), jnp.int32),
                       pltpu.VMEM((TQ,), jnp.float32),
                       pltpu.VMEM((TQ,), jnp.float32)]
        + [pltpu.SemaphoreType.DMA] * 5,
        compiler_params=pltpu.CompilerParams(needs_layout_passes=False),
    )
    def k(t0, t1, t2, t3, t4, t5, te, x0, x1, x2, x3, x4, x5, x6,
          f0h, f1h, f2h, out, rowv, rowe, ix0, ix1, oq0, oq1,
          rsem, ls0, ls1, ws0, ws1):
        tabs = (t0, t1, t2, t3, t4, t5, te)
        idxs = (x0, x1, x2, x3, x4, x5, x6)
        fhs = (f0h, f1h, f2h)
        wid = lax.axis_index("s") * NC + lax.axis_index("c")

        def drain(sem, buf):
            src = idxs[0] if buf.dtype == jnp.int32 else out
            pltpu.make_async_copy(src.at[pl.ds(0, TQ)], buf, sem).wait()

        def fire_idx(ci, q, buf, sem):
            pltpu.async_copy(
                idxs[ci].at[pl.ds(pl.multiple_of(q * TQ, TQ), TQ)], buf, sem)

        def flush(c, d, q, buf, sem):
            for lr in range(LCH):
                off = ((q * LCH + lr) * C + c) * D * N + d * N
                pltpu.async_copy(buf.at[pl.ds(lr * N, N)],
                                 out.at[pl.ds(pl.multiple_of(off, N), N)],
                                 sem)

        def gather(row, iq, oq):
            def body(i, _):
                for u in range(8):
                    sl = pl.ds(i * 128 + u * 16, 16)
                    oq[sl] = plsc.load_gather(row, [iq[sl]])
                return 0

            lax.fori_loop(0, TQ // 128, body, 0)

        def gather_task(ci, c, d):
            tab, row = tabs[ci], (rowv if ci < 6 else rowe)
            pltpu.async_copy(tab.at[d], row, rsem).wait()
            fire_idx(ci, 0, ix0, ls0)

            def hbody(h, _):
                q0, q1 = 2 * h, 2 * h + 1
                fire_idx(ci, q1, ix1, ls1)
                drain(ls0, ix0)

                @pl.when(h > 0)
                def _():
                    drain(ws0, oq0)
                gather(row, ix0, oq0)
                flush(c, d, q0, oq0, ws0)
                nxt = lax.rem(q1 + 1, NQ)
                fire_idx(ci, nxt, ix0, ls0)
                drain(ls1, ix1)

                @pl.when(h > 0)
                def _():
                    drain(ws1, oq1)
                gather(row, ix1, oq1)
                flush(c, d, q1, oq1, ws1)
                return 0

            lax.fori_loop(0, NQ // 2, hbody, 0)
            drain(ls0, ix0)
            drain(ws0, oq0)
            drain(ws1, oq1)

        def copy_task(fi, c, d):
            fh = fhs[fi]

            def qbody(q, _):
                pltpu.async_copy(
                    fh.at[pl.ds(pl.multiple_of(d * NT + q * TQ, TQ), TQ)],
                    oq0, ls0).wait()
                flush(c, d, q, oq0, ws0)
                drain(ws0, oq0)
                return 0

            lax.fori_loop(0, NQ, qbody, 0)

        def rbody(rep, _):
            d = wid * 2 + rep
            for ci, c in enumerate((0, 1, 2, 3, 4, 5, 8)):
                gather_task(ci, c, d)
            for fi, c in enumerate((6, 7, 9)):
                copy_task(fi, c, d)
            return 0

        lax.fori_loop(0, 2, rbody, 0)

    return k(*tabsT, *idxF, *foursF)


def kernel(discrete_0, table_d0, discrete_1, table_d1, discrete_2, table_d2,
           discrete_3, table_d3, discrete_4, table_d4, discrete_5, table_d5,
           entity, table_e,
           cont0_values, cont0_indicators, W_cont0, b_cont0,
           cont1_values, cont1_indicators, W_cont1, b_cont1,
           temp_values, temp_indicators, W_temp, b_temp):
    Z = jnp.zeros((2 * NWF, D), jnp.float32)
    Wbig = jnp.concatenate([
        jnp.concatenate([W_cont0, Z, Z], axis=1),
        jnp.concatenate([Z, W_cont1, Z], axis=1),
        jnp.concatenate([Z, Z, W_temp], axis=1)], axis=0)

    def tok(a):
        return a.T.reshape(NT // 128, 128)

    f0, f1, f2 = _fourier_tc(
        tok(cont0_values), tok(cont0_indicators),
        tok(cont1_values), tok(cont1_indicators),
        tok(temp_values), tok(temp_indicators),
        Wbig, b_cont0.reshape(D, 1), b_cont1.reshape(D, 1),
        b_temp.reshape(D, 1))

    idxF = [a.T.reshape(NT) for a in
            (discrete_0, discrete_1, discrete_2, discrete_3, discrete_4,
             discrete_5, entity)]
    tabsT = [t.T for t in
             (table_d0, table_d1, table_d2, table_d3, table_d4, table_d5,
              table_e)]
    foursF = [f.reshape(D * NT) for f in (f0, f1, f2)]

    out = _sc_scatter(tabsT, idxF, foursF)
    return out.reshape(L, C, D, N).transpose(3, 0, 1, 2)


# speedup vs baseline: 2.3915x; 1.0566x over previous
import functools
import math

import jax
import jax.numpy as jnp
from jax import lax
from jax.experimental import pallas as pl
from jax.experimental.pallas import tpu as pltpu
from jax.experimental.pallas import tpu_sc as plsc

N, L, D = 1024, 50, 64
NT = N * L
N_BANDS, OFFSET = 8, 3
NWF = N_BANDS + OFFSET + 1
C = 10
VD = 100004
VE = 2052
OUTSZ = L * C * D * N

NC, NS = 2, 16
NWORK = NC * NS

LCH = 5
TQ = LCH * N
NQ = L // LCH

G = 16
TBLK = G * 128

_W0 = math.pi * (2.0 ** float(-N_BANDS))


def _fourier_tc(v0, i0, v1, i1, v2, i2, Wbig, b0, b1, b2):
    def body(v0r, i0r, v1r, i1r, v2r, i2r, Wr, b0r, b1r, b2r, o0r, o1r, o2r):
        rows = []
        for vr, ir in ((v0r, i0r), (v1r, i1r), (v2r, i2r)):
            x = vr[...] - ir[...]
            angles = [x * (_W0 * 2.0 ** k) for k in range(NWF)]
            rows.extend([jnp.sin(a) for a in angles])
            rows.extend([jnp.cos(a) for a in angles])
        S = jnp.stack(rows)
        W = Wr[...]
        for r in range(G):
            o = lax.dot_general(W, S[:, r, :], (((0,), (0,)), ((), ())),
                                preferred_element_type=jnp.float32)
            sl = pl.ds(r * 128, 128)
            o0r[:, sl] = o[0:D, :] + b0r[...]
            o1r[:, sl] = o[D:2 * D, :] + b1r[...]
            o2r[:, sl] = o[2 * D:3 * D, :] + b2r[...]

    blk = pl.BlockSpec((G, 128), lambda i: (i, 0))
    wspec = pl.BlockSpec((6 * NWF, 3 * D), lambda i: (0, 0))
    bspec = pl.BlockSpec((D, 1), lambda i: (0, 0))
    outspec = pl.BlockSpec((D, TBLK), lambda i: (0, i))
    return pl.pallas_call(
        body,
        grid=(NT // TBLK,),
        in_specs=[blk] * 6 + [wspec, bspec, bspec, bspec],
        out_specs=[outspec] * 3,
        out_shape=[jax.ShapeDtypeStruct((D, NT), jnp.float32)] * 3,
    )(v0, i0, v1, i1, v2, i2, Wbig, b0, b1, b2)


def _sc_scatter(tabsT, idxF, foursF):
    mesh = plsc.VectorSubcoreMesh(core_axis_name="c", subcore_axis_name="s",
                                  num_cores=NC, num_subcores=NS)

    @functools.partial(
        pl.kernel,
        out_type=jax.ShapeDtypeStruct((OUTSZ,), jnp.float32),
        mesh=mesh,
        scratch_types=[pltpu.VMEM((VD,), jnp.float32),
                       pltpu.VMEM((VE,), jnp.float32),
                       pltpu.VMEM((TQ,), jnp.int32),
                       pltpu.VMEM((TQ,), jnp.int32),
                       pltpu.VMEM((TQ,), jnp.float32),
                       pltpu.VMEM((TQ,), jnp.float32)]
        + [pltpu.SemaphoreType.DMA] * 5,
        compiler_params=pltpu.CompilerParams(needs_layout_passes=False),
    )
    def k(t0, t1, t2, t3, t4, t5, te, x0, x1, x2, x3, x4, x5, x6,
          f0h, f1h, f2h, out, rowv, rowe, ix0, ix1, oq0, oq1,
          rsem, ls0, ls1, ws0, ws1):
        tabs = (t0, t1, t2, t3, t4, t5, te)
        idxs = (x0, x1, x2, x3, x4, x5, x6)
        fhs = (f0h, f1h, f2h)
        wid = lax.axis_index("s") * NC + lax.axis_index("c")

        def drain(sem, buf):
            src = idxs[0] if buf.dtype == jnp.int32 else out
            pltpu.make_async_copy(src.at[pl.ds(0, TQ)], buf, sem).wait()

        def fire_idx(ci, q, buf, sem):
            pltpu.async_copy(
                idxs[ci].at[pl.ds(pl.multiple_of(q * TQ, TQ), TQ)], buf, sem)

        def flush(c, d, q, buf, sem):
            for lr in range(LCH):
                off = ((q * LCH + lr) * C + c) * D * N + d * N
                pltpu.async_copy(buf.at[pl.ds(lr * N, N)],
                                 out.at[pl.ds(pl.multiple_of(off, N), N)],
                                 sem)

        def gather(row, iq, oq):
            def body(i, _):
                for u in range(8):
                    sl = pl.ds(i * 128 + u * 16, 16)
                    oq[sl] = plsc.load_gather(row, [iq[sl]])
                return 0

            lax.fori_loop(0, TQ // 128, body, 0)

        def gather_task(ci, c, d):
            tab, row = tabs[ci], (rowv if ci < 6 else rowe)
            pltpu.async_copy(tab.at[d], row, rsem).wait()
            fire_idx(ci, 0, ix0, ls0)

            def hbody(h, _):
                q0, q1 = 2 * h, 2 * h + 1
                fire_idx(ci, q1, ix1, ls1)
                drain(ls0, ix0)

                @pl.when(h > 0)
                def _():
                    drain(ws0, oq0)
                gather(row, ix0, oq0)
                flush(c, d, q0, oq0, ws0)
                nxt = lax.rem(q1 + 1, NQ)
                fire_idx(ci, nxt, ix0, ls0)
                drain(ls1, ix1)

                @pl.when(h > 0)
                def _():
                    drain(ws1, oq1)
                gather(row, ix1, oq1)
                flush(c, d, q1, oq1, ws1)
                return 0

            lax.fori_loop(0, NQ // 2, hbody, 0)
            drain(ls0, ix0)
            drain(ws0, oq0)
            drain(ws1, oq1)

        def copy_task(fi, c, d):
            fh = fhs[fi]

            def qbody(q, _):
                pltpu.async_copy(
                    fh.at[d, pl.ds(pl.multiple_of(q * TQ, TQ), TQ)],
                    oq0, ls0).wait()
                flush(c, d, q, oq0, ws0)
                drain(ws0, oq0)
                return 0

            lax.fori_loop(0, NQ, qbody, 0)

        def rbody(rep, _):
            d = wid * 2 + rep
            for ci, c in enumerate((0, 1, 2, 3, 4, 5, 8)):
                gather_task(ci, c, d)
            for fi, c in enumerate((6, 7, 9)):
                copy_task(fi, c, d)
            return 0

        lax.fori_loop(0, 2, rbody, 0)

    return k(*tabsT, *idxF, *foursF)


def kernel(discrete_0, table_d0, discrete_1, table_d1, discrete_2, table_d2,
           discrete_3, table_d3, discrete_4, table_d4, discrete_5, table_d5,
           entity, table_e,
           cont0_values, cont0_indicators, W_cont0, b_cont0,
           cont1_values, cont1_indicators, W_cont1, b_cont1,
           temp_values, temp_indicators, W_temp, b_temp):
    Z = jnp.zeros((2 * NWF, D), jnp.float32)
    Wbig = jnp.concatenate([
        jnp.concatenate([W_cont0, Z, Z], axis=1),
        jnp.concatenate([Z, W_cont1, Z], axis=1),
        jnp.concatenate([Z, Z, W_temp], axis=1)], axis=0)

    def tok(a):
        return a.T.reshape(NT // 128, 128)

    f0, f1, f2 = _fourier_tc(
        tok(cont0_values), tok(cont0_indicators),
        tok(cont1_values), tok(cont1_indicators),
        tok(temp_values), tok(temp_indicators),
        Wbig, b_cont0.reshape(D, 1), b_cont1.reshape(D, 1),
        b_temp.reshape(D, 1))

    idxF = [a.T.reshape(NT) for a in
            (discrete_0, discrete_1, discrete_2, discrete_3, discrete_4,
             discrete_5, entity)]
    tabsT = [t.T for t in
             (table_d0, table_d1, table_d2, table_d3, table_d4, table_d5,
              table_e)]
    foursF = (f0, f1, f2)

    out = _sc_scatter(tabsT, idxF, foursF)
    return out.reshape(L, C, D, N).transpose(3, 0, 1, 2)


# speedup vs baseline: 2.7828x; 1.1636x over previous
import functools
import math

import jax
import jax.numpy as jnp
from jax import lax
from jax.experimental import pallas as pl
from jax.experimental.pallas import tpu as pltpu
from jax.experimental.pallas import tpu_sc as plsc

N, L, D = 1024, 50, 64
NT = N * L
N_BANDS, OFFSET = 8, 3
NWF = N_BANDS + OFFSET + 1
C = 10
VD = 100004
VE = 2052
OUTSZ = L * C * D * N

NC, NS = 2, 16
NWORK = NC * NS

LCH = 5
TQ = LCH * N
NQ = L // LCH

G = 16
TBLK = G * 128

_W0 = math.pi * (2.0 ** float(-N_BANDS))


def _fourier_tc(v0, i0, v1, i1, v2, i2, Wbig, b0, b1, b2):
    def body(v0r, i0r, v1r, i1r, v2r, i2r, Wr, b0r, b1r, b2r, o0r, o1r, o2r):
        rows = []
        for vr, ir in ((v0r, i0r), (v1r, i1r), (v2r, i2r)):
            x = vr[...] - ir[...]
            angles = [x * (_W0 * 2.0 ** k) for k in range(NWF)]
            rows.extend([jnp.sin(a) for a in angles])
            rows.extend([jnp.cos(a) for a in angles])
        S = jnp.stack(rows)
        W = Wr[...]
        for r in range(G):
            o = lax.dot_general(W, S[:, r, :], (((0,), (0,)), ((), ())),
                                preferred_element_type=jnp.float32)
            sl = pl.ds(r * 128, 128)
            o0r[:, sl] = o[0:D, :] + b0r[...]
            o1r[:, sl] = o[D:2 * D, :] + b1r[...]
            o2r[:, sl] = o[2 * D:3 * D, :] + b2r[...]

    blk = pl.BlockSpec((G, 128), lambda i: (i, 0))
    wspec = pl.BlockSpec((6 * NWF, 3 * D), lambda i: (0, 0))
    bspec = pl.BlockSpec((D, 1), lambda i: (0, 0))
    outspec = pl.BlockSpec((D, TBLK), lambda i: (0, i))
    return pl.pallas_call(
        body,
        grid=(NT // TBLK,),
        in_specs=[blk] * 6 + [wspec, bspec, bspec, bspec],
        out_specs=[outspec] * 3,
        out_shape=[jax.ShapeDtypeStruct((D, NT), jnp.float32)] * 3,
    )(v0, i0, v1, i1, v2, i2, Wbig, b0, b1, b2)


def _sc_scatter(tabsT, idxF, foursF):
    mesh = plsc.VectorSubcoreMesh(core_axis_name="c", subcore_axis_name="s",
                                  num_cores=NC, num_subcores=NS)

    @functools.partial(
        pl.kernel,
        out_type=jax.ShapeDtypeStruct((OUTSZ,), jnp.float32),
        mesh=mesh,
        scratch_types=[pltpu.VMEM((VD,), jnp.float32),
                       pltpu.VMEM((VE,), jnp.float32),
                       pltpu.VMEM((TQ,), jnp.int32),
                       pltpu.VMEM((TQ,), jnp.int32),
                       pltpu.VMEM((TQ,), jnp.float32),
                       pltpu.VMEM((TQ,), jnp.float32)]
        + [pltpu.SemaphoreType.DMA] * 5,
        compiler_params=pltpu.CompilerParams(needs_layout_passes=False),
    )
    def k(t0, t1, t2, t3, t4, t5, te, x0, x1, x2, x3, x4, x5, x6,
          f0h, f1h, f2h, out, rowv, rowe, ix0, ix1, oq0, oq1,
          rsem, ls0, ls1, ws0, ws1):
        tabs = (t0, t1, t2, t3, t4, t5, te)
        idxs = (x0, x1, x2, x3, x4, x5, x6)
        fhs = (f0h, f1h, f2h)
        wid = lax.axis_index("s") * NC + lax.axis_index("c")

        def drain(sem, buf):
            src = idxs[0] if buf.dtype == jnp.int32 else out
            pltpu.make_async_copy(src.at[pl.ds(0, TQ)], buf, sem).wait()

        def fire_idx(ci, q, buf, sem):
            pltpu.async_copy(
                idxs[ci].at[pl.ds(pl.multiple_of(q * TQ, TQ), TQ)], buf, sem)

        def flush(c, d, q, buf, sem):
            for lr in range(LCH):
                off = ((q * LCH + lr) * C + c) * D * N + d * N
                pltpu.async_copy(buf.at[pl.ds(lr * N, N)],
                                 out.at[pl.ds(pl.multiple_of(off, N), N)],
                                 sem)

        def gather(row, iq, oq):
            @plsc.parallel_loop(0, TQ // 16, unroll=8)
            def _(i):
                sl = pl.ds(i * 16, 16)
                oq[sl] = plsc.load_gather(row, [iq[sl]])

        def gather_task(ci, c, d):
            tab, row = tabs[ci], (rowv if ci < 6 else rowe)
            pltpu.async_copy(tab.at[d], row, rsem).wait()
            fire_idx(ci, 0, ix0, ls0)

            def hbody(h, _):
                q0, q1 = 2 * h, 2 * h + 1
                fire_idx(ci, q1, ix1, ls1)
                drain(ls0, ix0)

                @pl.when(h > 0)
                def _():
                    drain(ws0, oq0)
                gather(row, ix0, oq0)
                flush(c, d, q0, oq0, ws0)
                nxt = lax.rem(q1 + 1, NQ)
                fire_idx(ci, nxt, ix0, ls0)
                drain(ls1, ix1)

                @pl.when(h > 0)
                def _():
                    drain(ws1, oq1)
                gather(row, ix1, oq1)
                flush(c, d, q1, oq1, ws1)
                return 0

            lax.fori_loop(0, NQ // 2, hbody, 0)
            drain(ls0, ix0)
            drain(ws0, oq0)
            drain(ws1, oq1)

        def copy_task(fi, c, d):
            fh = fhs[fi]

            def qbody(q, _):
                pltpu.async_copy(
                    fh.at[d, pl.ds(pl.multiple_of(q * TQ, TQ), TQ)],
                    oq0, ls0).wait()
                flush(c, d, q, oq0, ws0)
                drain(ws0, oq0)
                return 0

            lax.fori_loop(0, NQ, qbody, 0)

        def rbody(rep, _):
            d = wid * 2 + rep
            for ci, c in enumerate((0, 1, 2, 3, 4, 5, 8)):
                gather_task(ci, c, d)
            for fi, c in enumerate((6, 7, 9)):
                copy_task(fi, c, d)
            return 0

        lax.fori_loop(0, 2, rbody, 0)

    return k(*tabsT, *idxF, *foursF)


def kernel(discrete_0, table_d0, discrete_1, table_d1, discrete_2, table_d2,
           discrete_3, table_d3, discrete_4, table_d4, discrete_5, table_d5,
           entity, table_e,
           cont0_values, cont0_indicators, W_cont0, b_cont0,
           cont1_values, cont1_indicators, W_cont1, b_cont1,
           temp_values, temp_indicators, W_temp, b_temp):
    Z = jnp.zeros((2 * NWF, D), jnp.float32)
    Wbig = jnp.concatenate([
        jnp.concatenate([W_cont0, Z, Z], axis=1),
        jnp.concatenate([Z, W_cont1, Z], axis=1),
        jnp.concatenate([Z, Z, W_temp], axis=1)], axis=0)

    def tok(a):
        return a.T.reshape(NT // 128, 128)

    f0, f1, f2 = _fourier_tc(
        tok(cont0_values), tok(cont0_indicators),
        tok(cont1_values), tok(cont1_indicators),
        tok(temp_values), tok(temp_indicators),
        Wbig, b_cont0.reshape(D, 1), b_cont1.reshape(D, 1),
        b_temp.reshape(D, 1))

    idxF = [a.T.reshape(NT) for a in
            (discrete_0, discrete_1, discrete_2, discrete_3, discrete_4,
             discrete_5, entity)]
    tabsT = [t.T for t in
             (table_d0, table_d1, table_d2, table_d3, table_d4, table_d5,
              table_e)]
    foursF = (f0, f1, f2)

    out = _sc_scatter(tabsT, idxF, foursF)
    return out.reshape(L, C, D, N).transpose(3, 0, 1, 2)


# speedup vs baseline: 2.8977x; 1.0413x over previous
import functools
import math

import jax
import jax.numpy as jnp
from jax import lax
from jax.experimental import pallas as pl
from jax.experimental.pallas import tpu as pltpu
from jax.experimental.pallas import tpu_sc as plsc

N, L, D = 1024, 50, 64
NT = N * L
N_BANDS, OFFSET = 8, 3
NWF = N_BANDS + OFFSET + 1
C = 10
VD = 100004
VE = 2052
OUTSZ = L * C * D * N

NC, NS = 2, 16
NWORK = NC * NS

LCH = 5
TQ = LCH * N
NQ = L // LCH

G = 16
TBLK = G * 128

_W0 = math.pi * (2.0 ** float(-N_BANDS))


def _fourier_tc(v0, i0, v1, i1, v2, i2, Wbig, b0, b1, b2):
    def body(v0r, i0r, v1r, i1r, v2r, i2r, Wr, b0r, b1r, b2r, o0r, o1r, o2r):
        rows = []
        for vr, ir in ((v0r, i0r), (v1r, i1r), (v2r, i2r)):
            x = vr[...] - ir[...]
            angles = [x * (_W0 * 2.0 ** k) for k in range(NWF)]
            rows.extend([jnp.sin(a) for a in angles])
            rows.extend([jnp.cos(a) for a in angles])
        S = jnp.stack(rows)
        W = Wr[...]
        for r in range(G):
            o = lax.dot_general(W, S[:, r, :], (((0,), (0,)), ((), ())),
                                preferred_element_type=jnp.float32)
            sl = pl.ds(r * 128, 128)
            o0r[:, sl] = o[0:D, :] + b0r[...]
            o1r[:, sl] = o[D:2 * D, :] + b1r[...]
            o2r[:, sl] = o[2 * D:3 * D, :] + b2r[...]

    blk = pl.BlockSpec((G, 128), lambda i: (i, 0))
    wspec = pl.BlockSpec((6 * NWF, 3 * D), lambda i: (0, 0))
    bspec = pl.BlockSpec((D, 1), lambda i: (0, 0))
    outspec = pl.BlockSpec((D, TBLK), lambda i: (0, i))
    return pl.pallas_call(
        body,
        grid=(NT // TBLK,),
        in_specs=[blk] * 6 + [wspec, bspec, bspec, bspec],
        out_specs=[outspec] * 3,
        out_shape=[jax.ShapeDtypeStruct((D, NT), jnp.float32)] * 3,
    )(v0, i0, v1, i1, v2, i2, Wbig, b0, b1, b2)


def _sc_scatter(tabsT, idxF, foursF):
    mesh = plsc.VectorSubcoreMesh(core_axis_name="c", subcore_axis_name="s",
                                  num_cores=NC, num_subcores=NS)

    @functools.partial(
        pl.kernel,
        out_type=jax.ShapeDtypeStruct((OUTSZ,), jnp.float32),
        mesh=mesh,
        scratch_types=[pltpu.VMEM((VD,), jnp.float32),
                       pltpu.VMEM((VE,), jnp.float32),
                       pltpu.VMEM((TQ,), jnp.int32),
                       pltpu.VMEM((TQ,), jnp.int32),
                       pltpu.VMEM((TQ,), jnp.float32),
                       pltpu.VMEM((TQ,), jnp.float32)]
        + [pltpu.SemaphoreType.DMA] * 5,
        compiler_params=pltpu.CompilerParams(needs_layout_passes=False),
    )
    def k(t0, t1, t2, t3, t4, t5, te, x0, x1, x2, x3, x4, x5, x6,
          f0h, f1h, f2h, out, rowv, rowe, ix0, ix1, oq0, oq1,
          rsem, ls0, ls1, ws0, ws1):
        tabs = (t0, t1, t2, t3, t4, t5, te)
        idxs = (x0, x1, x2, x3, x4, x5, x6)
        fhs = (f0h, f1h, f2h)
        wid = lax.axis_index("s") * NC + lax.axis_index("c")

        def drain(sem, buf):
            src = idxs[0] if buf.dtype == jnp.int32 else out
            pltpu.make_async_copy(src.at[pl.ds(0, TQ)], buf, sem).wait()

        def fire_idx(ci, q, buf, sem):
            pltpu.async_copy(
                idxs[ci].at[pl.ds(pl.multiple_of(q * TQ, TQ), TQ)], buf, sem)

        def flush(c, d, q, buf, sem):
            for lr in range(LCH):
                off = ((q * LCH + lr) * C + c) * D * N + d * N
                pltpu.async_copy(buf.at[pl.ds(lr * N, N)],
                                 out.at[pl.ds(pl.multiple_of(off, N), N)],
                                 sem)

        def gather(row, iq, oq):
            @plsc.parallel_loop(0, TQ // 16, unroll=8)
            def _(i):
                sl = pl.ds(i * 16, 16)
                oq[sl] = plsc.load_gather(row, [iq[sl]])

        def fire_row(ci, d):
            pltpu.async_copy(tabs[ci].at[d], rowv if ci < 6 else rowe, rsem)

        def gather_task(ci, c, d, prefetched=False):
            tab, row = tabs[ci], (rowv if ci < 6 else rowe)
            if not prefetched:
                fire_row(ci, d)
            pltpu.make_async_copy(tab.at[d], row, rsem).wait()
            fire_idx(ci, 0, ix0, ls0)

            def hbody(h, _):
                q0, q1 = 2 * h, 2 * h + 1
                fire_idx(ci, q1, ix1, ls1)
                drain(ls0, ix0)

                @pl.when(h > 0)
                def _():
                    drain(ws0, oq0)
                gather(row, ix0, oq0)
                flush(c, d, q0, oq0, ws0)
                nxt = lax.rem(q1 + 1, NQ)
                fire_idx(ci, nxt, ix0, ls0)
                drain(ls1, ix1)

                @pl.when(h > 0)
                def _():
                    drain(ws1, oq1)
                gather(row, ix1, oq1)
                flush(c, d, q1, oq1, ws1)
                return 0

            lax.fori_loop(0, NQ // 2, hbody, 0)
            drain(ls0, ix0)
            drain(ws0, oq0)
            drain(ws1, oq1)

        def copy_task(fi, c, d, pre=None):
            if pre is not None:
                fire_row(pre[0], d)
            fh = fhs[fi]

            def qbody(q, _):
                pltpu.async_copy(
                    fh.at[d, pl.ds(pl.multiple_of(q * TQ, TQ), TQ)],
                    oq0, ls0).wait()
                flush(c, d, q, oq0, ws0)
                drain(ws0, oq0)
                return 0

            lax.fori_loop(0, NQ, qbody, 0)

        def rbody(rep, _):
            d = wid * 2 + rep
            fire_row(6, d)
            gather_task(0, 0, d)
            copy_task(0, 6, d, pre=(1,))
            gather_task(1, 1, d, prefetched=True)
            copy_task(1, 7, d, pre=(2,))
            gather_task(2, 2, d, prefetched=True)
            copy_task(2, 9, d, pre=(3,))
            gather_task(3, 3, d, prefetched=True)
            fire_row(4, d)
            gather_task(6, 8, d, prefetched=True)
            gather_task(4, 4, d, prefetched=True)
            gather_task(5, 5, d)
            return 0

        lax.fori_loop(0, 2, rbody, 0)

    return k(*tabsT, *idxF, *foursF)


def kernel(discrete_0, table_d0, discrete_1, table_d1, discrete_2, table_d2,
           discrete_3, table_d3, discrete_4, table_d4, discrete_5, table_d5,
           entity, table_e,
           cont0_values, cont0_indicators, W_cont0, b_cont0,
           cont1_values, cont1_indicators, W_cont1, b_cont1,
           temp_values, temp_indicators, W_temp, b_temp):
    Z = jnp.zeros((2 * NWF, D), jnp.float32)
    Wbig = jnp.concatenate([
        jnp.concatenate([W_cont0, Z, Z], axis=1),
        jnp.concatenate([Z, W_cont1, Z], axis=1),
        jnp.concatenate([Z, Z, W_temp], axis=1)], axis=0)

    def tok(a):
        return a.T.reshape(NT // 128, 128)

    f0, f1, f2 = _fourier_tc(
        tok(cont0_values), tok(cont0_indicators),
        tok(cont1_values), tok(cont1_indicators),
        tok(temp_values), tok(temp_indicators),
        Wbig, b_cont0.reshape(D, 1), b_cont1.reshape(D, 1),
        b_temp.reshape(D, 1))

    idxF = [a.T.reshape(NT) for a in
            (discrete_0, discrete_1, discrete_2, discrete_3, discrete_4,
             discrete_5, entity)]
    tabsT = [t.T for t in
             (table_d0, table_d1, table_d2, table_d3, table_d4, table_d5,
              table_e)]
    foursF = (f0, f1, f2)

    out = _sc_scatter(tabsT, idxF, foursF)
    return out.reshape(L, C, D, N).transpose(3, 0, 1, 2)
